# baseline (speedup 1.0000x reference)
import functools

import jax
import jax.numpy as jnp
from jax import lax
from jax.experimental import pallas as pl
from jax.experimental.pallas import tpu as pltpu
from jax.experimental.pallas import tpu_sc as plsc


def _sc_info():
    info = plsc.get_sparse_core_info()
    return info.num_cores, info.num_subcores, info.num_lanes


def _make_pack_kernel(F, V, D):
    NC, NS, L = _sc_info()
    NW = NC * NS
    R = F * D
    assert R % 8 == 0
    NBLK = R // 8
    VW = V // 2
    VC = 4096
    NFULL = V // VC
    TAIL = V - NFULL * VC
    TAIL1 = (TAIL // 128) * 128
    TAIL2 = TAIL - TAIL1
    assert VC % 256 == 0 and NFULL % 2 == 0 and TAIL1 % 32 == 0

    mesh = plsc.VectorSubcoreMesh(core_axis_name="c", subcore_axis_name="s")

    @functools.partial(
        pl.kernel,
        mesh=mesh,
        compiler_params=pltpu.CompilerParams(
            use_tc_tiling_on_sc=True, needs_layout_passes=False
        ),
        out_type=jax.ShapeDtypeStruct((R * VW,), jnp.int32),
        scratch_types=[
            pltpu.VMEM((8, VC), jnp.float32),
            pltpu.VMEM((8, VC), jnp.float32),
            pltpu.VMEM((8 * VC // 2,), jnp.int32),
            pltpu.VMEM((8 * VC // 2,), jnp.int32),
            pltpu.VMEM((8, 32), jnp.float32),
            pltpu.SemaphoreType.DMA,
            pltpu.SemaphoreType.DMA,
            pltpu.SemaphoreType.DMA,
            pltpu.SemaphoreType.DMA,
        ],
    )
    def k(t2_hbm, out_hbm, in0, in1, o0, o1, tin, si0, si1, so0, so1):
        wid = lax.axis_index("s") * NC + lax.axis_index("c")
        ins = (in0, in1)
        outs = (o0, o1)
        isems = (si0, si1)
        osems = (so0, so1)

        def in_src(blk, c):
            return t2_hbm.at[pl.ds(blk * 8, 8), pl.ds(c * VC, VC)]

        def out_piece(blk, c, r, n):
            return out_hbm.at[pl.ds((blk * 8 + r) * VW + c * (VC // 2), n)]

        def pack_pair(a, b):
            return plsc.bitcast(
                plsc.pack(a, b, format=plsc.PackFormat.INTERLEAVED), jnp.int32
            )

        def pack_rows(sl, ncols, ostride):
            for r in range(8):
                def grp(g, carry, r=r, sl=sl):
                    loads = []
                    for u in range(8):
                        col = (g * 8 + u) * 32
                        loads.append(
                            (
                                col,
                                ins[sl][r, pl.ds(col, L)],
                                ins[sl][r, pl.ds(col + L, L)],
                            )
                        )
                    for col, a, b in loads:
                        outs[sl][pl.ds(r * ostride + col // 2, L)] = pack_pair(a, b)
                    return carry

                lax.fori_loop(0, ncols // 256, grp, 0)
                rem = (ncols % 256) // 32
                for q in range(rem):
                    col = (ncols // 256) * 256 + q * 32
                    a = ins[sl][r, pl.ds(col, L)]
                    b = ins[sl][r, pl.ds(col + L, L)]
                    outs[sl][pl.ds(r * ostride + col // 2, L)] = pack_pair(a, b)

        def drain_out(blk, c, sl):
            for r in range(8):
                pltpu.make_async_copy(
                    outs[sl].at[pl.ds(r * (VC // 2), VC // 2)],
                    out_piece(blk, c, r, VC // 2),
                    osems[sl],
                ).wait()

        NCHU = NFULL // 2
        assert NCHU % 2 == 0

        def do_unit(blk, half):
            c0 = half * NCHU
            pltpu.async_copy(in_src(blk, c0), ins[0], si0)

            def chunk_pair(kp, carry):
                for s2 in range(2):
                    k = kp * 2 + s2
                    c = c0 + k
                    sl = s2
                    pltpu.make_async_copy(in_src(blk, c), ins[sl], isems[sl]).wait()
                    if s2 == 0:
                        pltpu.async_copy(in_src(blk, c + 1), ins[1], isems[1])
                    else:
                        @pl.when(kp + 1 < NCHU // 2)
                        def _():
                            pltpu.async_copy(in_src(blk, c + 1), ins[0], isems[0])

                    @pl.when(kp >= 1)
                    def _():
                        drain_out(blk, c - 2, sl)

                    pack_rows(sl, VC, VC // 2)
                    for r in range(8):
                        pltpu.async_copy(
                            outs[sl].at[pl.ds(r * (VC // 2), VC // 2)],
                            out_piece(blk, c, r, VC // 2),
                            osems[sl],
                        )
                return carry

            lax.fori_loop(0, NCHU // 2, chunk_pair, 0)
            drain_out(blk, c0 + NCHU - 2, 0)
            drain_out(blk, c0 + NCHU - 1, 1)

            @pl.when(half == 1)
            def _tail():
                pltpu.sync_copy(
                    t2_hbm.at[pl.ds(blk * 8, 8), pl.ds(NFULL * VC, TAIL1)],
                    ins[0].at[pl.ds(0, 8), pl.ds(0, TAIL1)],
                )
                pack_rows(0, TAIL1, TAIL // 2)
                if TAIL2:
                    pltpu.sync_copy(
                        t2_hbm.at[pl.ds(blk * 8, 8), pl.ds(NFULL * VC + TAIL1, TAIL2)],
                        tin,
                    )
                    for r in range(8):
                        a = tin[r, pl.ds(0, L)]
                        b = tin[r, pl.ds(L, L)]
                        outs[0][pl.ds(r * (TAIL // 2) + TAIL1 // 2, L)] = (
                            pack_pair(a, b)
                        )
                for r in range(8):
                    pltpu.sync_copy(
                        outs[0].at[pl.ds(r * (TAIL // 2), TAIL // 2)],
                        out_hbm.at[
                            pl.ds((blk * 8 + r) * VW + NFULL * (VC // 2), TAIL // 2)
                        ],
                    )

        assert NW == 32
        lo = lax.shift_right_logical(wid * (2 * NBLK), 5)
        hi = lax.shift_right_logical((wid + 1) * (2 * NBLK), 5)

        def unit_step(u, carry):
            do_unit(
                lax.shift_right_logical(u, 1), lax.bitwise_and(u, 1)
            )
            return carry

        lax.fori_loop(lo, hi, unit_step, 0)

    return k


def _make_lookup_kernel(B, F, V, D):
    NC, NS, L = _sc_info()
    NW = NC * NS
    assert D == NW, "this kernel maps output dims 1:1 onto subcores"
    VW = V // 2
    assert VW % 8 == 0
    CH = 4096
    assert B % CH == 0
    NCH = B // CH
    GU = 32
    assert CH % (GU * L) == 0
    assert F % 2 == 0

    mesh = plsc.VectorSubcoreMesh(core_axis_name="c", subcore_axis_name="s")
    mask_hi = jnp.int32(-65536)

    @functools.partial(
        pl.kernel,
        mesh=mesh,
        compiler_params=pltpu.CompilerParams(
            use_tc_tiling_on_sc=False, needs_layout_passes=False
        ),
        out_type=jax.ShapeDtypeStruct((D * B,), jnp.float32),
        scratch_types=[
            pltpu.VMEM((VW,), jnp.int32),
            pltpu.VMEM((VW,), jnp.int32),
            pltpu.VMEM((CH,), jnp.int32),
            pltpu.VMEM((CH,), jnp.int32),
            pltpu.VMEM((B,), jnp.float32),
            pltpu.SemaphoreType.DMA,
            pltpu.SemaphoreType.DMA,
            pltpu.SemaphoreType.DMA,
            pltpu.SemaphoreType.DMA,
        ],
    )
    def k(x_hbm, tbl_hbm, out_hbm, v0, v1, i0, i1, acc, sv0, sv1, si0, si1):
        wid = lax.axis_index("s") * NC + lax.axis_index("c")
        islots = (i0, i1)
        isems = (si0, si1)

        def v_src(f):
            return tbl_hbm.at[pl.ds((f * D + wid) * VW, VW)]

        def i_src(f, ch):
            return x_hbm.at[pl.ds(f * B + ch * CH, CH)]

        zeros = jnp.zeros((L,), jnp.float32)

        def zero_body(i, carry):
            acc[pl.ds(i * L, L)] = zeros
            return carry

        lax.fori_loop(0, B // L, zero_body, 0)

        pltpu.async_copy(v_src(0), v0, sv0)
        pltpu.async_copy(v_src(1), v1, sv1)

        def field_step(f, vs, vsem):
            pltpu.make_async_copy(v_src(f), vs, vsem).wait()
            pltpu.async_copy(i_src(f, 0), islots[0], isems[0])
            for ch in range(NCH):
                sl = ch % 2
                pltpu.make_async_copy(i_src(f, ch), islots[sl], isems[sl]).wait()
                if ch + 1 < NCH:
                    nsl = (ch + 1) % 2
                    pltpu.async_copy(i_src(f, ch + 1), islots[nsl], isems[nsl])
                idxr = islots[sl]
                base_b = ch * CH

                def group_step(g, carry):
                    for u in range(GU):
                        off = (g * GU + u) * L
                        v = idxr[pl.ds(off, L)]
                        pair = lax.bitwise_or(
                            lax.shift_left(lax.shift_right_logical(v, 5), 4),
                            lax.bitwise_and(v, 15),
                        )
                        parity = lax.bitwise_and(
                            lax.shift_right_logical(v, 4), 1
                        )
                        w = plsc.load_gather(vs, [pair])
                        sh = lax.shift_left(lax.bitwise_xor(parity, 1), 4)
                        valbits = lax.bitwise_and(lax.shift_left(w, sh), mask_hi)
                        val = plsc.bitcast(valbits, jnp.float32)
                        asl = pl.ds(base_b + off, L)
                        acc[asl] = acc[asl] + val
                    return carry

                lax.fori_loop(0, CH // (GU * L), group_step, 0)

            @pl.when(f + 2 < F)
            def _():
                pltpu.async_copy(v_src(f + 2), vs, vsem)

        def outer(fo, carry):
            field_step(2 * fo, v0, sv0)
            field_step(2 * fo + 1, v1, sv1)
            return carry

        lax.fori_loop(0, F // 2, outer, 0)

        pltpu.sync_copy(acc, out_hbm.at[pl.ds(wid * B, B)])

    return k


def kernel(x, tables):
    B, F = x.shape
    Ft, V, D = tables.shape
    assert Ft == F
    t2 = tables.transpose(0, 2, 1).reshape(F * D, V)
    packed = _make_pack_kernel(F, V, D)(t2)
    x_flat = x.astype(jnp.int32).T.reshape(F * B)
    out_flat = _make_lookup_kernel(B, F, V, D)(x_flat, packed)
    return out_flat.reshape(D, B).T


# speedup vs baseline: 1.3638x; 1.3638x over previous
import functools

import jax
import jax.numpy as jnp
from jax import lax
from jax.experimental import pallas as pl
from jax.experimental.pallas import tpu as pltpu
from jax.experimental.pallas import tpu_sc as plsc


def _sc_info():
    info = plsc.get_sparse_core_info()
    return info.num_cores, info.num_subcores, info.num_lanes


def _make_pack_kernel(F, V, D):
    NC, NS, L = _sc_info()
    NW = NC * NS
    R = F * D
    assert R % 8 == 0
    NBLK = R // 8
    VW = V // 2
    VC = 4096
    NFULL = V // VC
    TAIL = V - NFULL * VC
    TAIL1 = (TAIL // 128) * 128
    TAIL2 = TAIL - TAIL1
    assert VC % 256 == 0 and NFULL % 2 == 0 and TAIL1 % 32 == 0

    mesh = plsc.VectorSubcoreMesh(core_axis_name="c", subcore_axis_name="s")

    @functools.partial(
        pl.kernel,
        mesh=mesh,
        compiler_params=pltpu.CompilerParams(
            use_tc_tiling_on_sc=True, needs_layout_passes=False
        ),
        out_type=jax.ShapeDtypeStruct((R * VW,), jnp.int32),
        scratch_types=[
            pltpu.VMEM((8, VC), jnp.float32),
            pltpu.VMEM((8, VC), jnp.float32),
            pltpu.VMEM((8 * VC // 2,), jnp.int32),
            pltpu.VMEM((8 * VC // 2,), jnp.int32),
            pltpu.VMEM((8, 32), jnp.float32),
            pltpu.SemaphoreType.DMA,
            pltpu.SemaphoreType.DMA,
            pltpu.SemaphoreType.DMA,
            pltpu.SemaphoreType.DMA,
        ],
    )
    def k(t2_hbm, out_hbm, in0, in1, o0, o1, tin, si0, si1, so0, so1):
        wid = lax.axis_index("s") * NC + lax.axis_index("c")
        ins = (in0, in1)
        outs = (o0, o1)
        isems = (si0, si1)
        osems = (so0, so1)

        def in_src(blk, c):
            return t2_hbm.at[pl.ds(blk * 8, 8), pl.ds(c * VC, VC)]

        def out_piece(blk, c, r, n):
            return out_hbm.at[pl.ds((blk * 8 + r) * VW + c * (VC // 2), n)]

        def pack_pair(a, b):
            return plsc.bitcast(
                plsc.pack(a, b, format=plsc.PackFormat.INTERLEAVED), jnp.int32
            )

        def pack_rows(sl, ncols, ostride):
            for r in range(8):
                def grp(g, carry, r=r, sl=sl):
                    loads = []
                    for u in range(8):
                        col = (g * 8 + u) * 32
                        loads.append(
                            (
                                col,
                                ins[sl][r, pl.ds(col, L)],
                                ins[sl][r, pl.ds(col + L, L)],
                            )
                        )
                    for col, a, b in loads:
                        outs[sl][pl.ds(r * ostride + col // 2, L)] = pack_pair(a, b)
                    return carry

                lax.fori_loop(0, ncols // 256, grp, 0)
                rem = (ncols % 256) // 32
                for q in range(rem):
                    col = (ncols // 256) * 256 + q * 32
                    a = ins[sl][r, pl.ds(col, L)]
                    b = ins[sl][r, pl.ds(col + L, L)]
                    outs[sl][pl.ds(r * ostride + col // 2, L)] = pack_pair(a, b)

        def drain_out(blk, c, sl):
            for r in range(8):
                pltpu.make_async_copy(
                    outs[sl].at[pl.ds(r * (VC // 2), VC // 2)],
                    out_piece(blk, c, r, VC // 2),
                    osems[sl],
                ).wait()

        NCHU = NFULL // 2
        assert NCHU % 2 == 0

        def do_unit(blk, half):
            c0 = half * NCHU
            pltpu.async_copy(in_src(blk, c0), ins[0], si0)

            def chunk_pair(kp, carry):
                for s2 in range(2):
                    k = kp * 2 + s2
                    c = c0 + k
                    sl = s2
                    pltpu.make_async_copy(in_src(blk, c), ins[sl], isems[sl]).wait()
                    if s2 == 0:
                        pltpu.async_copy(in_src(blk, c + 1), ins[1], isems[1])
                    else:
                        @pl.when(kp + 1 < NCHU // 2)
                        def _():
                            pltpu.async_copy(in_src(blk, c + 1), ins[0], isems[0])

                    @pl.when(kp >= 1)
                    def _():
                        drain_out(blk, c - 2, sl)

                    pack_rows(sl, VC, VC // 2)
                    for r in range(8):
                        pltpu.async_copy(
                            outs[sl].at[pl.ds(r * (VC // 2), VC // 2)],
                            out_piece(blk, c, r, VC // 2),
                            osems[sl],
                        )
                return carry

            lax.fori_loop(0, NCHU // 2, chunk_pair, 0)
            drain_out(blk, c0 + NCHU - 2, 0)
            drain_out(blk, c0 + NCHU - 1, 1)

            @pl.when(half == 1)
            def _tail():
                pltpu.sync_copy(
                    t2_hbm.at[pl.ds(blk * 8, 8), pl.ds(NFULL * VC, TAIL1)],
                    ins[0].at[pl.ds(0, 8), pl.ds(0, TAIL1)],
                )
                pack_rows(0, TAIL1, TAIL // 2)
                if TAIL2:
                    pltpu.sync_copy(
                        t2_hbm.at[pl.ds(blk * 8, 8), pl.ds(NFULL * VC + TAIL1, TAIL2)],
                        tin,
                    )
                    for r in range(8):
                        a = tin[r, pl.ds(0, L)]
                        b = tin[r, pl.ds(L, L)]
                        outs[0][pl.ds(r * (TAIL // 2) + TAIL1 // 2, L)] = (
                            pack_pair(a, b)
                        )
                for r in range(8):
                    pltpu.sync_copy(
                        outs[0].at[pl.ds(r * (TAIL // 2), TAIL // 2)],
                        out_hbm.at[
                            pl.ds((blk * 8 + r) * VW + NFULL * (VC // 2), TAIL // 2)
                        ],
                    )

        assert NW == 32
        lo = lax.shift_right_logical(wid * (2 * NBLK), 5)
        hi = lax.shift_right_logical((wid + 1) * (2 * NBLK), 5)

        def unit_step(u, carry):
            do_unit(
                lax.shift_right_logical(u, 1), lax.bitwise_and(u, 1)
            )
            return carry

        lax.fori_loop(lo, hi, unit_step, 0)

    return k


def _make_lookup_kernel(B, F, V, D):
    NC, NS, L = _sc_info()
    NW = NC * NS
    assert D == NW, "this kernel maps output dims 1:1 onto subcores"
    VW = V // 2
    assert VW % 8 == 0
    CH = 4096
    assert B % CH == 0
    NCH = B // CH
    GU = 16
    assert CH % (GU * L) == 0
    assert F % 2 == 0

    mesh = plsc.VectorSubcoreMesh(core_axis_name="c", subcore_axis_name="s")
    mask_hi = jnp.int32(-65536)

    @functools.partial(
        pl.kernel,
        mesh=mesh,
        compiler_params=pltpu.CompilerParams(
            use_tc_tiling_on_sc=False, needs_layout_passes=False
        ),
        out_type=jax.ShapeDtypeStruct((D * B,), jnp.float32),
        scratch_types=[
            pltpu.VMEM((VW,), jnp.int32),
            pltpu.VMEM((VW,), jnp.int32),
            pltpu.VMEM((CH,), jnp.int32),
            pltpu.VMEM((CH,), jnp.int32),
            pltpu.VMEM((B,), jnp.float32),
            pltpu.SemaphoreType.DMA,
            pltpu.SemaphoreType.DMA,
            pltpu.SemaphoreType.DMA,
            pltpu.SemaphoreType.DMA,
        ],
    )
    def k(x_hbm, tbl_hbm, out_hbm, v0, v1, i0, i1, acc, sv0, sv1, si0, si1):
        wid = lax.axis_index("s") * NC + lax.axis_index("c")
        islots = (i0, i1)
        isems = (si0, si1)

        def v_src(f):
            return tbl_hbm.at[pl.ds((f * D + wid) * VW, VW)]

        def i_src(f, ch):
            return x_hbm.at[pl.ds(f * B + ch * CH, CH)]

        zeros = jnp.zeros((L,), jnp.float32)

        def zero_body(i, carry):
            acc[pl.ds(i * L, L)] = zeros
            return carry

        lax.fori_loop(0, B // L, zero_body, 0)

        pltpu.async_copy(v_src(0), v0, sv0)
        pltpu.async_copy(v_src(1), v1, sv1)

        def field_step(f, vs, vsem):
            pltpu.make_async_copy(v_src(f), vs, vsem).wait()
            pltpu.async_copy(i_src(f, 0), islots[0], isems[0])
            for ch in range(NCH):
                sl = ch % 2
                pltpu.make_async_copy(i_src(f, ch), islots[sl], isems[sl]).wait()
                if ch + 1 < NCH:
                    nsl = (ch + 1) % 2
                    pltpu.async_copy(i_src(f, ch + 1), islots[nsl], isems[nsl])
                idxr = islots[sl]
                base_b = ch * CH

                def group_step(g, carry):
                    for q in range(GU // 4):
                        offs = [(g * GU + q * 4 + t) * L for t in range(4)]
                        vsamp = [idxr[pl.ds(o, L)] for o in offs]
                        pairs = [
                            lax.bitwise_or(
                                lax.shift_left(lax.shift_right_logical(v, 5), 4),
                                lax.bitwise_and(v, 15),
                            )
                            for v in vsamp
                        ]
                        shs = [
                            lax.shift_left(
                                lax.bitwise_xor(
                                    lax.bitwise_and(
                                        lax.shift_right_logical(v, 4), 1
                                    ),
                                    1,
                                ),
                                4,
                            )
                            for v in vsamp
                        ]
                        ws = [plsc.load_gather(vs, [p]) for p in pairs]
                        vals = [
                            plsc.bitcast(
                                lax.bitwise_and(lax.shift_left(w, sh), mask_hi),
                                jnp.float32,
                            )
                            for w, sh in zip(ws, shs)
                        ]
                        for o, val in zip(offs, vals):
                            asl = pl.ds(base_b + o, L)
                            acc[asl] = acc[asl] + val
                    return carry

                lax.fori_loop(0, CH // (GU * L), group_step, 0)

            @pl.when(f + 2 < F)
            def _():
                pltpu.async_copy(v_src(f + 2), vs, vsem)

        def outer(fo, carry):
            field_step(2 * fo, v0, sv0)
            field_step(2 * fo + 1, v1, sv1)
            return carry

        lax.fori_loop(0, F // 2, outer, 0)

        pltpu.sync_copy(acc, out_hbm.at[pl.ds(wid * B, B)])

    return k


def kernel(x, tables):
    B, F = x.shape
    Ft, V, D = tables.shape
    assert Ft == F
    t2 = tables.transpose(0, 2, 1).reshape(F * D, V)
    packed = _make_pack_kernel(F, V, D)(t2)
    x_flat = x.astype(jnp.int32).T.reshape(F * B)
    out_flat = _make_lookup_kernel(B, F, V, D)(x_flat, packed)
    return out_flat.reshape(D, B).T


# speedup vs baseline: 1.3821x; 1.0135x over previous
import functools

import jax
import jax.numpy as jnp
from jax import lax
from jax.experimental import pallas as pl
from jax.experimental.pallas import tpu as pltpu
from jax.experimental.pallas import tpu_sc as plsc


def _sc_info():
    info = plsc.get_sparse_core_info()
    return info.num_cores, info.num_subcores, info.num_lanes


def _make_pack_kernel(F, V, D):
    NC, NS, L = _sc_info()
    NW = NC * NS
    R = F * D
    assert R % 8 == 0
    NBLK = R // 8
    VW = V // 2
    VC = 4096
    NFULL = V // VC
    TAIL = V - NFULL * VC
    TAIL1 = (TAIL // 128) * 128
    TAIL2 = TAIL - TAIL1
    assert VC % 256 == 0 and NFULL % 2 == 0 and TAIL1 % 32 == 0

    mesh = plsc.VectorSubcoreMesh(core_axis_name="c", subcore_axis_name="s")

    @functools.partial(
        pl.kernel,
        mesh=mesh,
        compiler_params=pltpu.CompilerParams(
            use_tc_tiling_on_sc=True, needs_layout_passes=False
        ),
        out_type=jax.ShapeDtypeStruct((R * VW,), jnp.int32),
        scratch_types=[
            pltpu.VMEM((8, VC), jnp.float32),
            pltpu.VMEM((8, VC), jnp.float32),
            pltpu.VMEM((8 * VC // 2,), jnp.int32),
            pltpu.VMEM((8 * VC // 2,), jnp.int32),
            pltpu.VMEM((8, 32), jnp.float32),
            pltpu.SemaphoreType.DMA,
            pltpu.SemaphoreType.DMA,
            pltpu.SemaphoreType.DMA,
            pltpu.SemaphoreType.DMA,
        ],
    )
    def k(t2_hbm, out_hbm, in0, in1, o0, o1, tin, si0, si1, so0, so1):
        wid = lax.axis_index("s") * NC + lax.axis_index("c")
        ins = (in0, in1)
        outs = (o0, o1)
        isems = (si0, si1)
        osems = (so0, so1)

        def in_src(blk, c):
            return t2_hbm.at[pl.ds(blk * 8, 8), pl.ds(c * VC, VC)]

        def out_piece(blk, c, r, n):
            return out_hbm.at[pl.ds((blk * 8 + r) * VW + c * (VC // 2), n)]

        def pack_pair(a, b):
            return plsc.bitcast(
                plsc.pack(a, b, format=plsc.PackFormat.INTERLEAVED), jnp.int32
            )

        def pack_rows(sl, ncols, ostride):
            for r in range(8):
                def grp(g, carry, r=r, sl=sl):
                    loads = []
                    for u in range(8):
                        col = (g * 8 + u) * 32
                        loads.append(
                            (
                                col,
                                ins[sl][r, pl.ds(col, L)],
                                ins[sl][r, pl.ds(col + L, L)],
                            )
                        )
                    for col, a, b in loads:
                        outs[sl][pl.ds(r * ostride + col // 2, L)] = pack_pair(a, b)
                    return carry

                lax.fori_loop(0, ncols // 256, grp, 0)
                rem = (ncols % 256) // 32
                for q in range(rem):
                    col = (ncols // 256) * 256 + q * 32
                    a = ins[sl][r, pl.ds(col, L)]
                    b = ins[sl][r, pl.ds(col + L, L)]
                    outs[sl][pl.ds(r * ostride + col // 2, L)] = pack_pair(a, b)

        def drain_out(blk, c, sl):
            for r in range(8):
                pltpu.make_async_copy(
                    outs[sl].at[pl.ds(r * (VC // 2), VC // 2)],
                    out_piece(blk, c, r, VC // 2),
                    osems[sl],
                ).wait()

        NCHU = NFULL // 2
        assert NCHU % 2 == 0

        def do_unit(blk, half):
            c0 = half * NCHU
            pltpu.async_copy(in_src(blk, c0), ins[0], si0)

            def chunk_pair(kp, carry):
                for s2 in range(2):
                    k = kp * 2 + s2
                    c = c0 + k
                    sl = s2
                    pltpu.make_async_copy(in_src(blk, c), ins[sl], isems[sl]).wait()
                    if s2 == 0:
                        pltpu.async_copy(in_src(blk, c + 1), ins[1], isems[1])
                    else:
                        @pl.when(kp + 1 < NCHU // 2)
                        def _():
                            pltpu.async_copy(in_src(blk, c + 1), ins[0], isems[0])

                    @pl.when(kp >= 1)
                    def _():
                        drain_out(blk, c - 2, sl)

                    pack_rows(sl, VC, VC // 2)
                    for r in range(8):
                        pltpu.async_copy(
                            outs[sl].at[pl.ds(r * (VC // 2), VC // 2)],
                            out_piece(blk, c, r, VC // 2),
                            osems[sl],
                        )
                return carry

            lax.fori_loop(0, NCHU // 2, chunk_pair, 0)
            drain_out(blk, c0 + NCHU - 2, 0)
            drain_out(blk, c0 + NCHU - 1, 1)

            @pl.when(half == 1)
            def _tail():
                pltpu.sync_copy(
                    t2_hbm.at[pl.ds(blk * 8, 8), pl.ds(NFULL * VC, TAIL1)],
                    ins[0].at[pl.ds(0, 8), pl.ds(0, TAIL1)],
                )
                pack_rows(0, TAIL1, TAIL // 2)
                if TAIL2:
                    pltpu.sync_copy(
                        t2_hbm.at[pl.ds(blk * 8, 8), pl.ds(NFULL * VC + TAIL1, TAIL2)],
                        tin,
                    )
                    for r in range(8):
                        a = tin[r, pl.ds(0, L)]
                        b = tin[r, pl.ds(L, L)]
                        outs[0][pl.ds(r * (TAIL // 2) + TAIL1 // 2, L)] = (
                            pack_pair(a, b)
                        )
                for r in range(8):
                    pltpu.sync_copy(
                        outs[0].at[pl.ds(r * (TAIL // 2), TAIL // 2)],
                        out_hbm.at[
                            pl.ds((blk * 8 + r) * VW + NFULL * (VC // 2), TAIL // 2)
                        ],
                    )

        assert NW == 32
        lo = lax.shift_right_logical(wid * (2 * NBLK), 5)
        hi = lax.shift_right_logical((wid + 1) * (2 * NBLK), 5)

        def unit_step(u, carry):
            do_unit(
                lax.shift_right_logical(u, 1), lax.bitwise_and(u, 1)
            )
            return carry

        lax.fori_loop(lo, hi, unit_step, 0)

    return k


def _make_lookup_kernel(B, F, V, D):
    NC, NS, L = _sc_info()
    NW = NC * NS
    assert D == NW, "this kernel maps output dims 1:1 onto subcores"
    VW = V // 2
    assert VW % 8 == 0
    CH = 4096
    assert B % CH == 0
    NCH = B // CH
    GU = 16
    assert CH % (GU * L) == 0
    assert F % 2 == 0

    mesh = plsc.VectorSubcoreMesh(core_axis_name="c", subcore_axis_name="s")
    mask_hi = jnp.int32(-65536)

    @functools.partial(
        pl.kernel,
        mesh=mesh,
        compiler_params=pltpu.CompilerParams(
            use_tc_tiling_on_sc=False, needs_layout_passes=False
        ),
        out_type=jax.ShapeDtypeStruct((D * B,), jnp.float32),
        scratch_types=[
            pltpu.VMEM((VW,), jnp.int32),
            pltpu.VMEM((VW,), jnp.int32),
            pltpu.VMEM((CH,), jnp.int32),
            pltpu.VMEM((CH,), jnp.int32),
            pltpu.VMEM((B,), jnp.float32),
            pltpu.SemaphoreType.DMA,
            pltpu.SemaphoreType.DMA,
            pltpu.SemaphoreType.DMA,
            pltpu.SemaphoreType.DMA,
        ],
    )
    def k(x_hbm, tbl_hbm, out_hbm, v0, v1, i0, i1, acc, sv0, sv1, si0, si1):
        wid = lax.axis_index("s") * NC + lax.axis_index("c")
        islots = (i0, i1)
        isems = (si0, si1)

        def v_src(f):
            return tbl_hbm.at[pl.ds((f * D + wid) * VW, VW)]

        def i_src(f, ch):
            return x_hbm.at[pl.ds(f * B + ch * CH, CH)]

        zeros = jnp.zeros((L,), jnp.float32)

        def zero_body(i, carry):
            acc[pl.ds(i * L, L)] = zeros
            return carry

        lax.fori_loop(0, B // L, zero_body, 0)

        pltpu.async_copy(v_src(0), v0, sv0)
        pltpu.async_copy(v_src(1), v1, sv1)

        def field_step(f, vs, vsem):
            pltpu.make_async_copy(v_src(f), vs, vsem).wait()
            pltpu.async_copy(i_src(f, 0), islots[0], isems[0])
            for ch in range(NCH):
                sl = ch % 2
                pltpu.make_async_copy(i_src(f, ch), islots[sl], isems[sl]).wait()
                if ch + 1 < NCH:
                    nsl = (ch + 1) % 2
                    pltpu.async_copy(i_src(f, ch + 1), islots[nsl], isems[nsl])
                idxr = islots[sl]
                base_b = ch * CH

                def group_step(g, carry):
                    for q in range(GU // 8):
                        offs = [(g * GU + q * 8 + t) * L for t in range(8)]
                        vsamp = [idxr[pl.ds(o, L)] for o in offs]
                        pairs = [
                            lax.bitwise_or(
                                lax.shift_left(lax.shift_right_logical(v, 5), 4),
                                lax.bitwise_and(v, 15),
                            )
                            for v in vsamp
                        ]
                        shs = [
                            lax.shift_left(
                                lax.bitwise_xor(
                                    lax.bitwise_and(
                                        lax.shift_right_logical(v, 4), 1
                                    ),
                                    1,
                                ),
                                4,
                            )
                            for v in vsamp
                        ]
                        ws = [plsc.load_gather(vs, [p]) for p in pairs]
                        vals = [
                            plsc.bitcast(
                                lax.bitwise_and(lax.shift_left(w, sh), mask_hi),
                                jnp.float32,
                            )
                            for w, sh in zip(ws, shs)
                        ]
                        for o, val in zip(offs, vals):
                            asl = pl.ds(base_b + o, L)
                            acc[asl] = acc[asl] + val
                    return carry

                lax.fori_loop(0, CH // (GU * L), group_step, 0)

            @pl.when(f + 2 < F)
            def _():
                pltpu.async_copy(v_src(f + 2), vs, vsem)

        def outer(fo, carry):
            field_step(2 * fo, v0, sv0)
            field_step(2 * fo + 1, v1, sv1)
            return carry

        lax.fori_loop(0, F // 2, outer, 0)

        pltpu.sync_copy(acc, out_hbm.at[pl.ds(wid * B, B)])

    return k


def kernel(x, tables):
    B, F = x.shape
    Ft, V, D = tables.shape
    assert Ft == F
    t2 = tables.transpose(0, 2, 1).reshape(F * D, V)
    packed = _make_pack_kernel(F, V, D)(t2)
    x_flat = x.astype(jnp.int32).T.reshape(F * B)
    out_flat = _make_lookup_kernel(B, F, V, D)(x_flat, packed)
    return out_flat.reshape(D, B).T


# speedup vs baseline: 1.3854x; 1.0024x over previous
import functools

import jax
import jax.numpy as jnp
from jax import lax
from jax.experimental import pallas as pl
from jax.experimental.pallas import tpu as pltpu
from jax.experimental.pallas import tpu_sc as plsc


def _sc_info():
    info = plsc.get_sparse_core_info()
    return info.num_cores, info.num_subcores, info.num_lanes


def _make_pack_kernel(F, V, D):
    NC, NS, L = _sc_info()
    NW = NC * NS
    R = F * D
    assert R % 8 == 0
    NBLK = R // 8
    VW = V // 2
    VC = 4096
    NFULL = V // VC
    TAIL = V - NFULL * VC
    TAIL1 = (TAIL // 128) * 128
    TAIL2 = TAIL - TAIL1
    assert VC % 256 == 0 and NFULL % 2 == 0 and TAIL1 % 32 == 0

    mesh = plsc.VectorSubcoreMesh(core_axis_name="c", subcore_axis_name="s")

    @functools.partial(
        pl.kernel,
        mesh=mesh,
        compiler_params=pltpu.CompilerParams(
            use_tc_tiling_on_sc=True, needs_layout_passes=False
        ),
        out_type=jax.ShapeDtypeStruct((R * VW,), jnp.int32),
        scratch_types=[
            pltpu.VMEM((8, VC), jnp.float32),
            pltpu.VMEM((8, VC), jnp.float32),
            pltpu.VMEM((8 * VC // 2,), jnp.int32),
            pltpu.VMEM((8 * VC // 2,), jnp.int32),
            pltpu.VMEM((8, 32), jnp.float32),
            pltpu.SemaphoreType.DMA,
            pltpu.SemaphoreType.DMA,
            pltpu.SemaphoreType.DMA,
            pltpu.SemaphoreType.DMA,
        ],
    )
    def k(t2_hbm, out_hbm, in0, in1, o0, o1, tin, si0, si1, so0, so1):
        wid = lax.axis_index("s") * NC + lax.axis_index("c")
        ins = (in0, in1)
        outs = (o0, o1)
        isems = (si0, si1)
        osems = (so0, so1)

        def in_src(blk, c):
            return t2_hbm.at[pl.ds(blk * 8, 8), pl.ds(c * VC, VC)]

        def out_piece(blk, c, r, n):
            return out_hbm.at[pl.ds((blk * 8 + r) * VW + c * (VC // 2), n)]

        def pack_pair(a, b):
            return plsc.bitcast(
                plsc.pack(a, b, format=plsc.PackFormat.INTERLEAVED), jnp.int32
            )

        def pack_rows(sl, ncols, ostride):
            for r in range(8):
                def grp(g, carry, r=r, sl=sl):
                    loads = []
                    for u in range(8):
                        col = (g * 8 + u) * 32
                        loads.append(
                            (
                                col,
                                ins[sl][r, pl.ds(col, L)],
                                ins[sl][r, pl.ds(col + L, L)],
                            )
                        )
                    for col, a, b in loads:
                        outs[sl][pl.ds(r * ostride + col // 2, L)] = pack_pair(a, b)
                    return carry

                lax.fori_loop(0, ncols // 256, grp, 0)
                rem = (ncols % 256) // 32
                for q in range(rem):
                    col = (ncols // 256) * 256 + q * 32
                    a = ins[sl][r, pl.ds(col, L)]
                    b = ins[sl][r, pl.ds(col + L, L)]
                    outs[sl][pl.ds(r * ostride + col // 2, L)] = pack_pair(a, b)

        def drain_out(blk, c, sl):
            for r in range(8):
                pltpu.make_async_copy(
                    outs[sl].at[pl.ds(r * (VC // 2), VC // 2)],
                    out_piece(blk, c, r, VC // 2),
                    osems[sl],
                ).wait()

        NCHU = NFULL // 2
        assert NCHU % 2 == 0

        def do_unit(blk, half):
            c0 = half * NCHU
            pltpu.async_copy(in_src(blk, c0), ins[0], si0)

            def chunk_pair(kp, carry):
                for s2 in range(2):
                    k = kp * 2 + s2
                    c = c0 + k
                    sl = s2
                    pltpu.make_async_copy(in_src(blk, c), ins[sl], isems[sl]).wait()
                    if s2 == 0:
                        pltpu.async_copy(in_src(blk, c + 1), ins[1], isems[1])
                    else:
                        @pl.when(kp + 1 < NCHU // 2)
                        def _():
                            pltpu.async_copy(in_src(blk, c + 1), ins[0], isems[0])

                    @pl.when(kp >= 1)
                    def _():
                        drain_out(blk, c - 2, sl)

                    pack_rows(sl, VC, VC // 2)
                    for r in range(8):
                        pltpu.async_copy(
                            outs[sl].at[pl.ds(r * (VC // 2), VC // 2)],
                            out_piece(blk, c, r, VC // 2),
                            osems[sl],
                        )
                return carry

            lax.fori_loop(0, NCHU // 2, chunk_pair, 0)
            drain_out(blk, c0 + NCHU - 2, 0)
            drain_out(blk, c0 + NCHU - 1, 1)

            @pl.when(half == 1)
            def _tail():
                pltpu.sync_copy(
                    t2_hbm.at[pl.ds(blk * 8, 8), pl.ds(NFULL * VC, TAIL1)],
                    ins[0].at[pl.ds(0, 8), pl.ds(0, TAIL1)],
                )
                pack_rows(0, TAIL1, TAIL // 2)
                if TAIL2:
                    pltpu.sync_copy(
                        t2_hbm.at[pl.ds(blk * 8, 8), pl.ds(NFULL * VC + TAIL1, TAIL2)],
                        tin,
                    )
                    for r in range(8):
                        a = tin[r, pl.ds(0, L)]
                        b = tin[r, pl.ds(L, L)]
                        outs[0][pl.ds(r * (TAIL // 2) + TAIL1 // 2, L)] = (
                            pack_pair(a, b)
                        )
                for r in range(8):
                    pltpu.sync_copy(
                        outs[0].at[pl.ds(r * (TAIL // 2), TAIL // 2)],
                        out_hbm.at[
                            pl.ds((blk * 8 + r) * VW + NFULL * (VC // 2), TAIL // 2)
                        ],
                    )

        assert NW == 32
        lo = lax.shift_right_logical(wid * (2 * NBLK), 5)
        hi = lax.shift_right_logical((wid + 1) * (2 * NBLK), 5)

        def unit_step(u, carry):
            do_unit(
                lax.shift_right_logical(u, 1), lax.bitwise_and(u, 1)
            )
            return carry

        lax.fori_loop(lo, hi, unit_step, 0)

    return k


def _make_lookup_kernel(B, F, V, D):
    NC, NS, L = _sc_info()
    NW = NC * NS
    assert D == NW, "this kernel maps output dims 1:1 onto subcores"
    VW = V // 2
    assert VW % 8 == 0
    CH = 4096
    assert B % CH == 0
    NCH = B // CH
    GU = 32
    assert CH % (GU * L) == 0
    assert F % 2 == 0

    mesh = plsc.VectorSubcoreMesh(core_axis_name="c", subcore_axis_name="s")
    mask_hi = jnp.int32(-65536)

    @functools.partial(
        pl.kernel,
        mesh=mesh,
        compiler_params=pltpu.CompilerParams(
            use_tc_tiling_on_sc=False, needs_layout_passes=False
        ),
        out_type=jax.ShapeDtypeStruct((D * B,), jnp.float32),
        scratch_types=[
            pltpu.VMEM((VW,), jnp.int32),
            pltpu.VMEM((VW,), jnp.int32),
            pltpu.VMEM((CH,), jnp.int32),
            pltpu.VMEM((CH,), jnp.int32),
            pltpu.VMEM((B,), jnp.float32),
            pltpu.SemaphoreType.DMA,
            pltpu.SemaphoreType.DMA,
            pltpu.SemaphoreType.DMA,
            pltpu.SemaphoreType.DMA,
        ],
    )
    def k(x_hbm, tbl_hbm, out_hbm, v0, v1, i0, i1, acc, sv0, sv1, si0, si1):
        wid = lax.axis_index("s") * NC + lax.axis_index("c")
        islots = (i0, i1)
        isems = (si0, si1)

        def v_src(f):
            return tbl_hbm.at[pl.ds((f * D + wid) * VW, VW)]

        def i_src(f, ch):
            return x_hbm.at[pl.ds(f * B + ch * CH, CH)]

        zeros = jnp.zeros((L,), jnp.float32)

        def zero_body(i, carry):
            acc[pl.ds(i * L, L)] = zeros
            return carry

        lax.fori_loop(0, B // L, zero_body, 0)

        pltpu.async_copy(v_src(0), v0, sv0)
        pltpu.async_copy(v_src(1), v1, sv1)

        def field_step(f, vs, vsem):
            pltpu.make_async_copy(v_src(f), vs, vsem).wait()
            pltpu.async_copy(i_src(f, 0), islots[0], isems[0])
            for ch in range(NCH):
                sl = ch % 2
                pltpu.make_async_copy(i_src(f, ch), islots[sl], isems[sl]).wait()
                if ch + 1 < NCH:
                    nsl = (ch + 1) % 2
                    pltpu.async_copy(i_src(f, ch + 1), islots[nsl], isems[nsl])
                idxr = islots[sl]
                base_b = ch * CH

                def group_step(g, carry):
                    for q in range(GU // 8):
                        offs = [(g * GU + q * 8 + t) * L for t in range(8)]
                        vsamp = [idxr[pl.ds(o, L)] for o in offs]
                        pairs = [
                            lax.bitwise_or(
                                lax.shift_left(lax.shift_right_logical(v, 5), 4),
                                lax.bitwise_and(v, 15),
                            )
                            for v in vsamp
                        ]
                        shs = [
                            lax.shift_left(
                                lax.bitwise_xor(
                                    lax.bitwise_and(
                                        lax.shift_right_logical(v, 4), 1
                                    ),
                                    1,
                                ),
                                4,
                            )
                            for v in vsamp
                        ]
                        ws = [plsc.load_gather(vs, [p]) for p in pairs]
                        vals = [
                            plsc.bitcast(
                                lax.bitwise_and(lax.shift_left(w, sh), mask_hi),
                                jnp.float32,
                            )
                            for w, sh in zip(ws, shs)
                        ]
                        for o, val in zip(offs, vals):
                            asl = pl.ds(base_b + o, L)
                            acc[asl] = acc[asl] + val
                    return carry

                lax.fori_loop(0, CH // (GU * L), group_step, 0)

            @pl.when(f + 2 < F)
            def _():
                pltpu.async_copy(v_src(f + 2), vs, vsem)

        def outer(fo, carry):
            field_step(2 * fo, v0, sv0)
            field_step(2 * fo + 1, v1, sv1)
            return carry

        lax.fori_loop(0, F // 2, outer, 0)

        pltpu.sync_copy(acc, out_hbm.at[pl.ds(wid * B, B)])

    return k


def kernel(x, tables):
    B, F = x.shape
    Ft, V, D = tables.shape
    assert Ft == F
    t2 = tables.transpose(0, 2, 1).reshape(F * D, V)
    packed = _make_pack_kernel(F, V, D)(t2)
    x_flat = x.astype(jnp.int32).T.reshape(F * B)
    out_flat = _make_lookup_kernel(B, F, V, D)(x_flat, packed)
    return out_flat.reshape(D, B).T
